# Initial kernel scaffold; baseline (speedup 1.0000x reference)
import functools

import jax
import jax.numpy as jnp
from jax import lax
from jax.experimental import pallas as pl
from jax.experimental.pallas import tpu as pltpu
from jax.experimental.pallas import tpu_sc as plsc

B = 16384
L = 50
D = 64
NW = 32
BPW = B // NW
NB = 8
NBL = NB * L
NCHUNK = BPW // NB

_mesh = plsc.VectorSubcoreMesh(core_axis_name="c", subcore_axis_name="s")


@functools.partial(
    pl.kernel,
    mesh=_mesh,
    out_type=jax.ShapeDtypeStruct((B, D), jnp.float32),
    scratch_types=[
        pltpu.VMEM((NBL,), jnp.int32),
        pltpu.VMEM((NBL, D), jnp.float32),
        pltpu.VMEM((1, D), jnp.float32),
        pltpu.VMEM((NB, D), jnp.float32),
        pltpu.SemaphoreType.DMA,
    ],
)
def _encode(w_hbm, seqs_hbm, out_hbm, idx_v, rows_v, w0_v, out_v, sem):
    wid = lax.axis_index("s") * 2 + lax.axis_index("c")
    base = wid * BPW
    pltpu.sync_copy(w_hbm.at[pl.ds(0, 1)], w0_v)
    w0 = [w0_v[0, pl.ds(k * 16, 16)] for k in range(4)]
    lanes = lax.iota(jnp.int32, 16)

    def chunk_body(c, carry):
        row0 = base + c * NB
        pltpu.sync_copy(seqs_hbm.at[pl.ds(row0 * L, NBL)], idx_v)
        pltpu.async_copy(w_hbm.at[idx_v], rows_v, sem).wait()

        def seq_body(j, jcarry):
            r0 = j * L
            acc = [jnp.zeros((16,), jnp.float32) for _ in range(4)]
            for l in range(L):
                for k in range(4):
                    acc[k] = acc[k] + rows_v[r0 + l, pl.ds(k * 16, 16)]
            z0 = idx_v[pl.ds(r0, 16)] == 0
            z1 = idx_v[pl.ds(r0 + 16, 16)] == 0
            z2 = idx_v[pl.ds(r0 + 32, 16)] == 0
            z3 = (idx_v[pl.ds(r0 + 34, 16)] == 0) & (lanes >= 14)
            nz = (plsc.all_reduce_population_count(z0)
                  + plsc.all_reduce_population_count(z1)
                  + plsc.all_reduce_population_count(z2)
                  + plsc.all_reduce_population_count(z3))
            nzf = nz.astype(jnp.float32)
            cnt = jnp.float32(L) - nzf
            scale = jnp.where(cnt > 0.5, 1.0 / cnt, jnp.zeros((16,), jnp.float32))
            for k in range(4):
                out_v[j, pl.ds(k * 16, 16)] = (acc[k] - nzf * w0[k]) * scale
            return jcarry

        lax.fori_loop(0, NB, seq_body, 0)
        pltpu.sync_copy(out_v, out_hbm.at[pl.ds(row0, NB)])
        return carry

    lax.fori_loop(0, NCHUNK, chunk_body, 0)


def kernel(W, seqs):
    return _encode(W, seqs.reshape(-1))


# speedup vs baseline: 2.3717x; 2.3717x over previous
import functools

import jax
import jax.numpy as jnp
from jax import lax
from jax.experimental import pallas as pl
from jax.experimental.pallas import tpu as pltpu
from jax.experimental.pallas import tpu_sc as plsc

B = 16384
L = 50
D = 64
NW = 32
BPW = B // NW
NB = 8
NBL = NB * L
NCHUNK = BPW // NB

_mesh = plsc.VectorSubcoreMesh(core_axis_name="c", subcore_axis_name="s")


@functools.partial(
    pl.kernel,
    mesh=_mesh,
    compiler_params=pltpu.CompilerParams(use_tc_tiling_on_sc=False),
    out_type=jax.ShapeDtypeStruct((B, D), jnp.float32),
    scratch_types=[
        pltpu.VMEM((NBL,), jnp.int32),
        pltpu.VMEM((NBL, D), jnp.float32),
        pltpu.VMEM((1, D), jnp.float32),
        pltpu.VMEM((NB, D), jnp.float32),
        pltpu.SemaphoreType.DMA,
    ],
)
def _encode(w_hbm, seqs_hbm, out_hbm, idx_v, rows_v, w0_v, out_v, sem):
    wid = lax.axis_index("s") * 2 + lax.axis_index("c")
    base = wid * BPW
    pltpu.sync_copy(w_hbm.at[pl.ds(0, 1)], w0_v)
    w0 = [w0_v[0, pl.ds(k * 16, 16)] for k in range(4)]
    lanes = lax.iota(jnp.int32, 16)

    def chunk_body(c, carry):
        row0 = base + c * NB
        pltpu.sync_copy(seqs_hbm.at[pl.ds(row0 * L, NBL)], idx_v)
        pltpu.async_copy(w_hbm.at[idx_v], rows_v, sem).wait()

        def seq_body(j, jcarry):
            r0 = j * L
            acc = [jnp.zeros((16,), jnp.float32) for _ in range(4)]
            for l in range(L):
                for k in range(4):
                    acc[k] = acc[k] + rows_v[r0 + l, pl.ds(k * 16, 16)]
            z0 = idx_v[pl.ds(r0, 16)] == 0
            z1 = idx_v[pl.ds(r0 + 16, 16)] == 0
            z2 = idx_v[pl.ds(r0 + 32, 16)] == 0
            z3 = (idx_v[pl.ds(r0 + 34, 16)] == 0) & (lanes >= 14)
            one = jnp.ones((16,), jnp.float32)
            zero = jnp.zeros((16,), jnp.float32)
            zf = (jnp.where(z0, one, zero) + jnp.where(z1, one, zero)
                  + jnp.where(z2, one, zero) + jnp.where(z3, one, zero))
            for sh in (8, 4, 2, 1):
                zf = zf + zf.at[lanes ^ sh].get(mode="promise_in_bounds")
            nzf = zf
            cnt = jnp.float32(L) - nzf
            scale = jnp.where(cnt > 0.5, 1.0 / cnt, jnp.zeros((16,), jnp.float32))
            for k in range(4):
                out_v[j, pl.ds(k * 16, 16)] = (acc[k] - nzf * w0[k]) * scale
            return jcarry

        lax.fori_loop(0, NB, seq_body, 0)
        pltpu.sync_copy(out_v, out_hbm.at[pl.ds(row0, NB)])
        return carry

    lax.fori_loop(0, NCHUNK, chunk_body, 0)


def kernel(W, seqs):
    return _encode(W, seqs.reshape(-1))


# speedup vs baseline: 2.7675x; 1.1669x over previous
import functools

import jax
import jax.numpy as jnp
from jax import lax
from jax.experimental import pallas as pl
from jax.experimental.pallas import tpu as pltpu
from jax.experimental.pallas import tpu_sc as plsc

B = 16384
L = 50
D = 64
NW = 32
BPW = B // NW
NB = 8
NBL = NB * L
NCHUNK = BPW // NB

_mesh = plsc.VectorSubcoreMesh(core_axis_name="c", subcore_axis_name="s")


@functools.partial(
    pl.kernel,
    mesh=_mesh,
    compiler_params=pltpu.CompilerParams(use_tc_tiling_on_sc=False),
    out_type=jax.ShapeDtypeStruct((B, D), jnp.float32),
    scratch_types=[
        pltpu.VMEM((NCHUNK, NBL), jnp.int32),
        pltpu.VMEM((NBL, D), jnp.float32),
        pltpu.VMEM((NBL, D), jnp.float32),
        pltpu.VMEM((1, D), jnp.float32),
        pltpu.VMEM((NB, D), jnp.float32),
        pltpu.VMEM((NB, D), jnp.float32),
        pltpu.SemaphoreType.DMA,
        pltpu.SemaphoreType.DMA,
        pltpu.SemaphoreType.DMA,
        pltpu.SemaphoreType.DMA,
    ],
)
def _encode(w_hbm, seqs_hbm, out_hbm, idx_all, rows0, rows1, w0_v,
            out0, out1, sg0, sg1, so0, so1):
    wid = lax.axis_index("s") * 2 + lax.axis_index("c")
    base = wid * BPW
    cbase = wid * NCHUNK
    pltpu.sync_copy(w_hbm.at[pl.ds(0, 1)], w0_v)
    pltpu.sync_copy(seqs_hbm.at[pl.ds(cbase, NCHUNK)], idx_all)
    w0 = [w0_v[0, pl.ds(k * 16, 16)] for k in range(4)]
    lanes = lax.iota(jnp.int32, 16)

    pltpu.async_copy(w_hbm.at[idx_all.at[0]], rows0, sg0)
    pltpu.async_copy(w_hbm.at[idx_all.at[1]], rows1, sg1)

    def compute_chunk(c, rows_v, out_v):
        def seq_body(j, jcarry):
            r0 = j * L
            acc = [jnp.zeros((16,), jnp.float32) for _ in range(4)]
            for l in range(L):
                for k in range(4):
                    acc[k] = acc[k] + rows_v[r0 + l, pl.ds(k * 16, 16)]
            z0 = idx_all[c, pl.ds(r0, 16)] == 0
            z1 = idx_all[c, pl.ds(r0 + 16, 16)] == 0
            z2 = idx_all[c, pl.ds(r0 + 32, 16)] == 0
            z3 = (idx_all[c, pl.ds(r0 + 34, 16)] == 0) & (lanes >= 14)
            one = jnp.ones((16,), jnp.float32)
            zero = jnp.zeros((16,), jnp.float32)
            zf = (jnp.where(z0, one, zero) + jnp.where(z1, one, zero)
                  + jnp.where(z2, one, zero) + jnp.where(z3, one, zero))
            for sh in (8, 4, 2, 1):
                zf = zf + zf.at[lanes ^ sh].get(mode="promise_in_bounds")
            cnt = jnp.float32(L) - zf
            scale = jnp.where(cnt > 0.5, 1.0 / cnt, zero)
            for k in range(4):
                out_v[j, pl.ds(k * 16, 16)] = (acc[k] - zf * w0[k]) * scale
            return jcarry
        lax.fori_loop(0, NB, seq_body, 0)

    def pair_body(i, carry):
        for par, rows_v, out_v, sg, so in ((0, rows0, out0, sg0, so0),
                                           (1, rows1, out1, sg1, so1)):
            c = 2 * i + par
            row0 = base + c * NB
            pltpu.make_async_copy(w_hbm.at[idx_all.at[c]], rows_v, sg).wait()

            @pl.when(i > 0)
            def _():
                pltpu.make_async_copy(out_v, out_hbm.at[pl.ds(row0, NB)],
                                      so).wait()

            compute_chunk(c, rows_v, out_v)

            @pl.when(c + 2 < NCHUNK)
            def _():
                pltpu.async_copy(w_hbm.at[idx_all.at[c + 2]], rows_v, sg)

            pltpu.async_copy(out_v, out_hbm.at[pl.ds(row0, NB)], so)
        return carry

    lax.fori_loop(0, NCHUNK // 2, pair_body, 0)
    pltpu.make_async_copy(out0, out_hbm.at[pl.ds(base, NB)], so0).wait()
    pltpu.make_async_copy(out1, out_hbm.at[pl.ds(base, NB)], so1).wait()


def kernel(W, seqs):
    return _encode(W, seqs.reshape(B // NB, NB * L))
